# baseline (speedup 1.0000x reference)
import functools

import jax
import jax.numpy as jnp
from jax import lax
from jax.experimental import pallas as pl
from jax.experimental.pallas import tpu as pltpu
from jax.experimental.pallas import tpu_sc as plsc

N = 10000
D = 128
E = 320000
L = 3

NC = 2
NS = 16
NW = NC * NS
E_PER_TILE = E // NW
CHUNK = 104
NCHUNK = -(-E_PER_TILE // CHUNK)
E_PT_PAD = NCHUNK * CHUNK
N_ACC = N + 8
ROWS_PER_TILE = 624
TAIL_ROWS = N_ACC - NS * ROWS_PER_TILE
DUMP_TAIL = N - NS * ROWS_PER_TILE
ZROWS = 24

NBUF = 3
GAHEAD = 2
SWAIT = NBUF - GAHEAD
IBUF = 8
IAHEAD = 6

_SC_MESH = plsc.VectorSubcoreMesh(core_axis_name="c", subcore_axis_name="s")


@functools.partial(
    pl.kernel,
    out_type=jax.ShapeDtypeStruct((NC, N, D), jnp.float32),
    mesh=_SC_MESH,
    scratch_types=[
        pltpu.VMEM_SHARED((N_ACC, D), jnp.float32),
        pltpu.VMEM((IBUF, CHUNK), jnp.int32),
        pltpu.VMEM((IBUF, CHUNK), jnp.int32),
        pltpu.VMEM((NBUF, CHUNK, D), jnp.float32),
        pltpu.VMEM((ZROWS, D), jnp.float32),
        pltpu.SemaphoreType.DMA,
        pltpu.SemaphoreType.DMA,
        pltpu.SemaphoreType.DMA,
    ],
)
def _sc_agg(h_hbm, src_hbm, dst_hbm, out_hbm, agg_sh, src_ring, dst_ring,
            rows_v, zbuf, isem, gsem, ssem):
    cid = lax.axis_index("c")
    sid = lax.axis_index("s")
    wid = cid * NS + sid
    ebase = wid * E_PT_PAD

    def _idx_issue(j, buf):
        e0 = ebase + j * CHUNK
        pltpu.async_copy(src_hbm.at[pl.ds(e0, CHUNK)], src_ring.at[buf], isem)
        pltpu.async_copy(dst_hbm.at[pl.ds(e0, CHUNK)], dst_ring.at[buf], isem)

    def _idx_wait(j, buf):
        e0 = ebase + j * CHUNK
        pltpu.make_async_copy(src_hbm.at[pl.ds(e0, CHUNK)], src_ring.at[buf],
                              isem).wait()
        pltpu.make_async_copy(dst_hbm.at[pl.ds(e0, CHUNK)], dst_ring.at[buf],
                              isem).wait()

    def _gather_issue(j, buf, ibuf):
        pltpu.async_copy(h_hbm.at[src_ring.at[ibuf]], rows_v.at[buf], gsem)

    def _gather_wait(j, buf, ibuf):
        pltpu.make_async_copy(h_hbm.at[src_ring.at[ibuf]], rows_v.at[buf],
                              gsem).wait()

    def _scatter_issue(j, buf, ibuf):
        pltpu.async_copy(rows_v.at[buf], agg_sh.at[dst_ring.at[ibuf]], ssem,
                         add=True)

    def _scatter_wait(j, buf, ibuf):
        pltpu.make_async_copy(rows_v.at[buf], agg_sh.at[dst_ring.at[ibuf]],
                              ssem).wait()

    for j in range(IAHEAD):
        _idx_issue(j, j)

    def _zrow(i, carry):
        for g in range(D // 16):
            zbuf[i, pl.ds(g * 16, 16)] = jnp.zeros((16,), jnp.float32)
        return carry

    lax.fori_loop(0, ZROWS, _zrow, 0)

    def _zcp(k, carry):
        pltpu.sync_copy(zbuf, agg_sh.at[pl.ds(sid * ROWS_PER_TILE + k * ZROWS,
                                              ZROWS)])
        return carry

    lax.fori_loop(0, ROWS_PER_TILE // ZROWS, _zcp, 0)

    @pl.when(sid == NS - 1)
    def _zero_tail():
        pltpu.sync_copy(zbuf, agg_sh.at[pl.ds(NS * ROWS_PER_TILE, TAIL_ROWS)])

    plsc.subcore_barrier()

    for j in range(GAHEAD):
        _idx_wait(j, j)
        _gather_issue(j, j, j)

    def _step(j, carry):
        b = lax.rem(j, NBUF)
        ib = lax.rem(j, IBUF)
        _gather_wait(j, b, ib)
        _scatter_issue(j, b, ib)

        @pl.when(j >= SWAIT)
        def _():
            jd = j - SWAIT
            _scatter_wait(jd, lax.rem(jd, NBUF), lax.rem(jd, IBUF))

        @pl.when(j + GAHEAD < NCHUNK)
        def _():
            jg = j + GAHEAD
            ibg = lax.rem(jg, IBUF)
            _idx_wait(jg, ibg)
            _gather_issue(jg, lax.rem(jg, NBUF), ibg)

        @pl.when(j + IAHEAD < NCHUNK)
        def _():
            ji = j + IAHEAD
            _idx_issue(ji, lax.rem(ji, IBUF))

        return carry

    lax.fori_loop(0, NCHUNK, _step, 0)
    for j in range(NCHUNK - SWAIT, NCHUNK):
        _scatter_wait(j, j % NBUF, j % IBUF)
    plsc.subcore_barrier()

    r0 = sid * ROWS_PER_TILE
    pltpu.sync_copy(agg_sh.at[pl.ds(r0, ROWS_PER_TILE)],
                    out_hbm.at[cid, pl.ds(r0, ROWS_PER_TILE)])

    @pl.when(sid == NS - 1)
    def _dump_tail():
        pltpu.sync_copy(agg_sh.at[pl.ds(NS * ROWS_PER_TILE, DUMP_TAIL)],
                        out_hbm.at[cid, pl.ds(NS * ROWS_PER_TILE, DUMP_TAIL)])


_ROWS_BLK = 1000
_GRID = N // _ROWS_BLK
_F32 = jnp.float32


def _proj_body(x_ref, w_ref, b_ref, o_ref):
    o_ref[...] = (jnp.dot(x_ref[...], w_ref[...], preferred_element_type=_F32)
                  + b_ref[...])


def _proj(x, w, bl):
    return pl.pallas_call(
        _proj_body,
        grid=(_GRID,),
        in_specs=[
            pl.BlockSpec((_ROWS_BLK, D), lambda i: (i, 0)),
            pl.BlockSpec((D, D), lambda i: (0, 0)),
            pl.BlockSpec((1, D), lambda i: (0, 0)),
        ],
        out_specs=pl.BlockSpec((_ROWS_BLK, D), lambda i: (i, 0)),
        out_shape=jax.ShapeDtypeStruct((N, D), _F32),
    )(x, w, bl.reshape(1, D))


def _combine_body(p_ref, hr_ref, wrel_ref, o_ref, *, relu):
    agg = p_ref[0] + p_ref[1]
    out = (jnp.dot(agg, wrel_ref[...], preferred_element_type=_F32)
           + hr_ref[...])
    if relu:
        out = jnp.maximum(out, 0.0)
    o_ref[...] = out


def _combine(p, hroot, wrel, relu):
    return pl.pallas_call(
        functools.partial(_combine_body, relu=relu),
        grid=(_GRID,),
        in_specs=[
            pl.BlockSpec((NC, _ROWS_BLK, D), lambda i: (0, i, 0)),
            pl.BlockSpec((_ROWS_BLK, D), lambda i: (i, 0)),
            pl.BlockSpec((D, D), lambda i: (0, 0)),
        ],
        out_specs=pl.BlockSpec((_ROWS_BLK, D), lambda i: (i, 0)),
        out_shape=jax.ShapeDtypeStruct((N, D), _F32),
    )(p, hroot, wrel)


def kernel(x, edge_index, edge_attr, batch, Wx, bx, W_rel, W_root, b):
    pad = E_PT_PAD - E_PER_TILE
    src = jnp.pad(edge_index[0].reshape(NW, E_PER_TILE),
                  ((0, 0), (0, pad))).reshape(-1)
    dst = jnp.pad(edge_index[1].reshape(NW, E_PER_TILE),
                  ((0, 0), (0, pad)), constant_values=N).reshape(-1)
    h = _proj(x, Wx, bx)
    for l in range(L):
        p = _sc_agg(h, src, dst)
        hroot = _proj(h, W_root[l], b[l])
        h = _combine(p, hroot, W_rel[l], relu=(l < L - 1))
    return h


# speedup vs baseline: 1.9762x; 1.9762x over previous
import functools

import jax
import jax.numpy as jnp
from jax import lax
from jax.experimental import pallas as pl
from jax.experimental.pallas import tpu as pltpu
from jax.experimental.pallas import tpu_sc as plsc

N = 10000
D = 128
E = 320000
L = 3

NC = 2
NS = 16
NW = NC * NS
E_PER_TILE = E // NW
CHUNK = 80
NCHUNK = E_PER_TILE // CHUNK
ROWS_PER_TILE = 624
TAIL_ROWS = N - NS * ROWS_PER_TILE
ZROWS = 48

NBUF = 4
GAHEAD = 3
SWAIT = NBUF - GAHEAD
IBUF = 8
IAHEAD = 6

_SC_MESH = plsc.VectorSubcoreMesh(core_axis_name="c", subcore_axis_name="s")


@functools.partial(
    pl.kernel,
    out_type=jax.ShapeDtypeStruct((NC, N, D), jnp.float32),
    mesh=_SC_MESH,
    scratch_types=[
        pltpu.VMEM_SHARED((N, D), jnp.float32),
        pltpu.VMEM((IBUF, CHUNK), jnp.int32),
        pltpu.VMEM((IBUF, CHUNK), jnp.int32),
        pltpu.VMEM((NBUF, CHUNK, D), jnp.float32),
        pltpu.VMEM((ZROWS, D), jnp.float32),
        pltpu.SemaphoreType.DMA,
        pltpu.SemaphoreType.DMA,
        pltpu.SemaphoreType.DMA,
    ],
)
def _sc_agg(h_hbm, src_hbm, dst_hbm, out_hbm, agg_sh, src_ring, dst_ring,
            rows_v, zbuf, isem, gsem, ssem):
    cid = lax.axis_index("c")
    sid = lax.axis_index("s")
    wid = cid * NS + sid
    ebase = wid * E_PER_TILE

    def _idx_issue(j, buf):
        e0 = ebase + j * CHUNK
        pltpu.async_copy(src_hbm.at[pl.ds(e0, CHUNK)], src_ring.at[buf], isem)
        pltpu.async_copy(dst_hbm.at[pl.ds(e0, CHUNK)], dst_ring.at[buf], isem)

    def _idx_wait(j, buf):
        e0 = ebase + j * CHUNK
        pltpu.make_async_copy(src_hbm.at[pl.ds(e0, CHUNK)], src_ring.at[buf],
                              isem).wait()
        pltpu.make_async_copy(dst_hbm.at[pl.ds(e0, CHUNK)], dst_ring.at[buf],
                              isem).wait()

    def _gather_issue(j, buf, ibuf):
        pltpu.async_copy(h_hbm.at[src_ring.at[ibuf]], rows_v.at[buf], gsem)

    def _gather_wait(j, buf, ibuf):
        pltpu.make_async_copy(h_hbm.at[src_ring.at[ibuf]], rows_v.at[buf],
                              gsem).wait()

    def _scatter_issue(j, buf, ibuf):
        pltpu.async_copy(rows_v.at[buf], agg_sh.at[dst_ring.at[ibuf]], ssem,
                         add=True)

    def _scatter_wait(j, buf, ibuf):
        pltpu.make_async_copy(rows_v.at[buf], agg_sh.at[dst_ring.at[ibuf]],
                              ssem).wait()

    for j in range(IAHEAD):
        _idx_issue(j, j)

    def _zrow(i, carry):
        for g in range(D // 16):
            zbuf[i, pl.ds(g * 16, 16)] = jnp.zeros((16,), jnp.float32)
        return carry

    lax.fori_loop(0, ZROWS, _zrow, 0)

    def _zcp(k, carry):
        pltpu.async_copy(zbuf, agg_sh.at[pl.ds(sid * ROWS_PER_TILE + k * ZROWS,
                                               ZROWS)], ssem)
        return carry

    lax.fori_loop(0, ROWS_PER_TILE // ZROWS, _zcp, 0)

    @pl.when(sid == NS - 1)
    def _zero_tail():
        pltpu.async_copy(zbuf.at[pl.ds(0, TAIL_ROWS)],
                         agg_sh.at[pl.ds(NS * ROWS_PER_TILE, TAIL_ROWS)], ssem)

    def _zcp_drain(k, carry):
        pltpu.make_async_copy(
            zbuf, agg_sh.at[pl.ds(sid * ROWS_PER_TILE + k * ZROWS, ZROWS)],
            ssem).wait()
        return carry

    lax.fori_loop(0, ROWS_PER_TILE // ZROWS, _zcp_drain, 0)

    @pl.when(sid == NS - 1)
    def _zero_tail_drain():
        pltpu.make_async_copy(
            zbuf.at[pl.ds(0, TAIL_ROWS)],
            agg_sh.at[pl.ds(NS * ROWS_PER_TILE, TAIL_ROWS)], ssem).wait()

    plsc.subcore_barrier()

    for j in range(GAHEAD):
        _idx_wait(j, j)
        _gather_issue(j, j, j)

    def _step(j, carry):
        b = lax.rem(j, NBUF)
        ib = lax.rem(j, IBUF)
        _gather_wait(j, b, ib)
        _scatter_issue(j, b, ib)

        @pl.when(j >= SWAIT)
        def _():
            jd = j - SWAIT
            _scatter_wait(jd, lax.rem(jd, NBUF), lax.rem(jd, IBUF))

        @pl.when(j + GAHEAD < NCHUNK)
        def _():
            jg = j + GAHEAD
            ibg = lax.rem(jg, IBUF)
            _idx_wait(jg, ibg)
            _gather_issue(jg, lax.rem(jg, NBUF), ibg)

        @pl.when(j + IAHEAD < NCHUNK)
        def _():
            ji = j + IAHEAD
            _idx_issue(ji, lax.rem(ji, IBUF))

        return carry

    lax.fori_loop(0, NCHUNK, _step, 0)
    for j in range(NCHUNK - SWAIT, NCHUNK):
        _scatter_wait(j, j % NBUF, j % IBUF)
    plsc.subcore_barrier()

    r0 = sid * ROWS_PER_TILE
    pltpu.sync_copy(agg_sh.at[pl.ds(r0, ROWS_PER_TILE)],
                    out_hbm.at[cid, pl.ds(r0, ROWS_PER_TILE)])

    @pl.when(sid == NS - 1)
    def _dump_tail():
        pltpu.sync_copy(agg_sh.at[pl.ds(NS * ROWS_PER_TILE, TAIL_ROWS)],
                        out_hbm.at[cid, pl.ds(NS * ROWS_PER_TILE, TAIL_ROWS)])


_ROWS_BLK = 1000
_GRID = N // _ROWS_BLK
_F32 = jnp.float32


def _proj_body(x_ref, w_ref, b_ref, o_ref):
    o_ref[...] = (jnp.dot(x_ref[...], w_ref[...], preferred_element_type=_F32)
                  + b_ref[...])


def _proj(x, w, bl):
    return pl.pallas_call(
        _proj_body,
        grid=(_GRID,),
        in_specs=[
            pl.BlockSpec((_ROWS_BLK, D), lambda i: (i, 0)),
            pl.BlockSpec((D, D), lambda i: (0, 0)),
            pl.BlockSpec((1, D), lambda i: (0, 0)),
        ],
        out_specs=pl.BlockSpec((_ROWS_BLK, D), lambda i: (i, 0)),
        out_shape=jax.ShapeDtypeStruct((N, D), _F32),
    )(x, w, bl.reshape(1, D))


def _combine_body(p_ref, hr_ref, wrel_ref, o_ref, *, relu):
    agg = p_ref[0] + p_ref[1]
    out = (jnp.dot(agg, wrel_ref[...], preferred_element_type=_F32)
           + hr_ref[...])
    if relu:
        out = jnp.maximum(out, 0.0)
    o_ref[...] = out


def _combine(p, hroot, wrel, relu):
    return pl.pallas_call(
        functools.partial(_combine_body, relu=relu),
        grid=(_GRID,),
        in_specs=[
            pl.BlockSpec((NC, _ROWS_BLK, D), lambda i: (0, i, 0)),
            pl.BlockSpec((_ROWS_BLK, D), lambda i: (i, 0)),
            pl.BlockSpec((D, D), lambda i: (0, 0)),
        ],
        out_specs=pl.BlockSpec((_ROWS_BLK, D), lambda i: (i, 0)),
        out_shape=jax.ShapeDtypeStruct((N, D), _F32),
    )(p, hroot, wrel)


def kernel(x, edge_index, edge_attr, batch, Wx, bx, W_rel, W_root, b):
    src = edge_index[0]
    dst = edge_index[1]
    h = _proj(x, Wx, bx)
    for l in range(L):
        p = _sc_agg(h, src, dst)
        hroot = _proj(h, W_root[l], b[l])
        h = _combine(p, hroot, W_rel[l], relu=(l < L - 1))
    return h


# speedup vs baseline: 1.9763x; 1.0001x over previous
import functools

import jax
import jax.numpy as jnp
from jax import lax
from jax.experimental import pallas as pl
from jax.experimental.pallas import tpu as pltpu
from jax.experimental.pallas import tpu_sc as plsc

N = 10000
D = 128
E = 320000
L = 3

NC = 2
NS = 16
NW = NC * NS
E_PER_TILE = E // NW
CHUNK = 80
NCHUNK = E_PER_TILE // CHUNK
ROWS_PER_TILE = 624
TAIL_ROWS = N - NS * ROWS_PER_TILE
ZROWS = 48

NBUF = 4
GAHEAD = 3
SWAIT = NBUF - GAHEAD
IBUF = 8
IAHEAD = 6

_SC_MESH = plsc.VectorSubcoreMesh(core_axis_name="c", subcore_axis_name="s")


@functools.partial(
    pl.kernel,
    out_type=jax.ShapeDtypeStruct((NC, N, D), jnp.float32),
    mesh=_SC_MESH,
    scratch_types=[
        pltpu.VMEM_SHARED((N, D), jnp.float32),
        pltpu.VMEM((IBUF, CHUNK), jnp.int32),
        pltpu.VMEM((IBUF, CHUNK), jnp.int32),
        pltpu.VMEM((NBUF, CHUNK, D), jnp.float32),
        pltpu.VMEM((ZROWS, D), jnp.float32),
        pltpu.SemaphoreType.DMA,
        pltpu.SemaphoreType.DMA,
        pltpu.SemaphoreType.DMA,
    ],
)
def _sc_agg(h_hbm, src_hbm, dst_hbm, out_hbm, agg_sh, src_ring, dst_ring,
            rows_v, zbuf, isem, gsem, ssem):
    cid = lax.axis_index("c")
    sid = lax.axis_index("s")
    wid = cid * NS + sid
    ebase = wid * E_PER_TILE

    def _idx_issue(j, buf):
        e0 = ebase + j * CHUNK
        pltpu.async_copy(src_hbm.at[pl.ds(e0, CHUNK)], src_ring.at[buf], isem)
        pltpu.async_copy(dst_hbm.at[pl.ds(e0, CHUNK)], dst_ring.at[buf], isem)

    def _idx_wait(j, buf):
        e0 = ebase + j * CHUNK
        pltpu.make_async_copy(src_hbm.at[pl.ds(e0, CHUNK)], src_ring.at[buf],
                              isem).wait()
        pltpu.make_async_copy(dst_hbm.at[pl.ds(e0, CHUNK)], dst_ring.at[buf],
                              isem).wait()

    def _gather_issue(j, buf, ibuf):
        pltpu.async_copy(h_hbm.at[src_ring.at[ibuf]], rows_v.at[buf], gsem)

    def _gather_wait(j, buf, ibuf):
        pltpu.make_async_copy(h_hbm.at[src_ring.at[ibuf]], rows_v.at[buf],
                              gsem).wait()

    def _scatter_issue(j, buf, ibuf):
        pltpu.async_copy(rows_v.at[buf], agg_sh.at[dst_ring.at[ibuf]], ssem,
                         add=True)

    def _scatter_wait(j, buf, ibuf):
        pltpu.make_async_copy(rows_v.at[buf], agg_sh.at[dst_ring.at[ibuf]],
                              ssem).wait()

    for j in range(IAHEAD):
        _idx_issue(j, j)

    def _zrow(i, carry):
        for g in range(D // 16):
            zbuf[i, pl.ds(g * 16, 16)] = jnp.zeros((16,), jnp.float32)
        return carry

    lax.fori_loop(0, ZROWS, _zrow, 0)

    def _zcp(k, carry):
        pltpu.async_copy(zbuf, agg_sh.at[pl.ds(sid * ROWS_PER_TILE + k * ZROWS,
                                               ZROWS)], ssem)
        return carry

    lax.fori_loop(0, ROWS_PER_TILE // ZROWS, _zcp, 0)

    @pl.when(sid == NS - 1)
    def _zero_tail():
        pltpu.async_copy(zbuf.at[pl.ds(0, TAIL_ROWS)],
                         agg_sh.at[pl.ds(NS * ROWS_PER_TILE, TAIL_ROWS)], ssem)

    def _zcp_drain(k, carry):
        pltpu.make_async_copy(
            zbuf, agg_sh.at[pl.ds(sid * ROWS_PER_TILE + k * ZROWS, ZROWS)],
            ssem).wait()
        return carry

    lax.fori_loop(0, ROWS_PER_TILE // ZROWS, _zcp_drain, 0)

    @pl.when(sid == NS - 1)
    def _zero_tail_drain():
        pltpu.make_async_copy(
            zbuf.at[pl.ds(0, TAIL_ROWS)],
            agg_sh.at[pl.ds(NS * ROWS_PER_TILE, TAIL_ROWS)], ssem).wait()

    plsc.subcore_barrier()

    for j in range(GAHEAD):
        _idx_wait(j, j)
        _gather_issue(j, j, j)

    def _step(j, carry):
        b = lax.rem(j, NBUF)
        ib = lax.rem(j, IBUF)
        _gather_wait(j, b, ib)
        _scatter_issue(j, b, ib)

        @pl.when(j >= SWAIT)
        def _():
            jd = j - SWAIT
            _scatter_wait(jd, lax.rem(jd, NBUF), lax.rem(jd, IBUF))

        @pl.when(j + GAHEAD < NCHUNK)
        def _():
            jg = j + GAHEAD
            ibg = lax.rem(jg, IBUF)
            _idx_wait(jg, ibg)
            _gather_issue(jg, lax.rem(jg, NBUF), ibg)

        @pl.when(j + IAHEAD < NCHUNK)
        def _():
            ji = j + IAHEAD
            _idx_issue(ji, lax.rem(ji, IBUF))

        return carry

    lax.fori_loop(0, NCHUNK, _step, 0)
    for j in range(NCHUNK - SWAIT, NCHUNK):
        _scatter_wait(j, j % NBUF, j % IBUF)
    plsc.subcore_barrier()

    r0 = sid * ROWS_PER_TILE
    pltpu.sync_copy(agg_sh.at[pl.ds(r0, ROWS_PER_TILE)],
                    out_hbm.at[cid, pl.ds(r0, ROWS_PER_TILE)])

    @pl.when(sid == NS - 1)
    def _dump_tail():
        pltpu.sync_copy(agg_sh.at[pl.ds(NS * ROWS_PER_TILE, TAIL_ROWS)],
                        out_hbm.at[cid, pl.ds(NS * ROWS_PER_TILE, TAIL_ROWS)])


_ROWS_BLK = 1000
_GRID = N // _ROWS_BLK
_F32 = jnp.float32


def _proj_body(x_ref, w_ref, b_ref, o_ref):
    o_ref[...] = (jnp.dot(x_ref[...], w_ref[...], preferred_element_type=_F32)
                  + b_ref[...])


def _proj(x, w, bl):
    return pl.pallas_call(
        _proj_body,
        grid=(_GRID,),
        in_specs=[
            pl.BlockSpec((_ROWS_BLK, D), lambda i: (i, 0)),
            pl.BlockSpec((D, D), lambda i: (0, 0)),
            pl.BlockSpec((1, D), lambda i: (0, 0)),
        ],
        out_specs=pl.BlockSpec((_ROWS_BLK, D), lambda i: (i, 0)),
        out_shape=jax.ShapeDtypeStruct((N, D), _F32),
    )(x, w, bl.reshape(1, D))


def _layer_body(p_ref, h_ref, wrel_ref, wroot_ref, b_ref, o_ref, *, relu):
    agg = p_ref[0] + p_ref[1]
    out = (jnp.dot(agg, wrel_ref[...], preferred_element_type=_F32)
           + jnp.dot(h_ref[...], wroot_ref[...], preferred_element_type=_F32)
           + b_ref[...])
    if relu:
        out = jnp.maximum(out, 0.0)
    o_ref[...] = out


def _layer(p, h, wrel, wroot, bl, relu):
    return pl.pallas_call(
        functools.partial(_layer_body, relu=relu),
        grid=(_GRID,),
        in_specs=[
            pl.BlockSpec((NC, _ROWS_BLK, D), lambda i: (0, i, 0)),
            pl.BlockSpec((_ROWS_BLK, D), lambda i: (i, 0)),
            pl.BlockSpec((D, D), lambda i: (0, 0)),
            pl.BlockSpec((D, D), lambda i: (0, 0)),
            pl.BlockSpec((1, D), lambda i: (0, 0)),
        ],
        out_specs=pl.BlockSpec((_ROWS_BLK, D), lambda i: (i, 0)),
        out_shape=jax.ShapeDtypeStruct((N, D), _F32),
    )(p, h, wrel, wroot, bl.reshape(1, D))


def kernel(x, edge_index, edge_attr, batch, Wx, bx, W_rel, W_root, b):
    src = edge_index[0]
    dst = edge_index[1]
    h = _proj(x, Wx, bx)
    for l in range(L):
        p = _sc_agg(h, src, dst)
        h = _layer(p, h, W_rel[l], W_root[l], b[l], relu=(l < L - 1))
    return h
